# baseline (speedup 1.0000x reference)
import functools

import jax
import jax.numpy as jnp
from jax.experimental import pallas as pl
from jax.experimental.pallas import tpu as pltpu

_PW = 256
_CH = 2048
_HOFF = 384


def _conv1_chunks(dfl, hflat_ref, w1, b1c, vm, c0, c1, C):
    for ci in range(c0, c1):
        n0 = ci * _CH
        acc = jnp.broadcast_to(b1c, (C, _CH))
        for dy in range(3):
            for dx in range(3):
                t = dy * 3 + dx
                off = dy * _PW + dx + n0
                acc = acc + w1[:, t][:, None] * dfl[:, off:off + _CH]
        h = jnp.where(vm, jnp.maximum(acc, 0.0), 0.0)
        hflat_ref[:, _HOFF + n0:_HOFF + n0 + _CH] = h


def _conv2_chunks(hflat_ref, feats_ref, w2_ref, b2c, c0, c1, C, W):
    RB = _CH // _PW
    for ci in range(c0, c1):
        n0 = ci * _CH
        facc = None
        for dy in range(3):
            for dx in range(3):
                t = dy * 3 + dx
                off = _HOFF - _PW - 1 + dy * _PW + dx + n0
                dres = jax.lax.dot_general(
                    w2_ref[t], hflat_ref[:, off:off + _CH],
                    (((1,), (0,)), ((), ())),
                    preferred_element_type=jnp.float32)
                facc = dres if facc is None else facc + dres
        feats = jnp.maximum(facc + b2c, 0.0)
        y0 = ci * RB
        feats_ref[:, y0:y0 + RB, :] = feats.reshape(C, RB, _PW)[:, :, :W]


def _argmin_rows(depth_ref, hyp, idx_ref, y0, y1, D, W):
    dch = depth_ref[0, 0, y0:y1, :]
    best = jnp.abs(dch - hyp[0])
    idx = jnp.zeros((y1 - y0, W), jnp.int32)
    for dd in range(1, D):
        diff = jnp.abs(dch - hyp[dd])
        take = diff < best
        best = jnp.where(take, diff, best)
        idx = jnp.where(take, dd, idx)
    idx_ref[y0:y1, :] = idx


def _zero_borders(hflat_ref, C, NP):
    hflat_ref[:, :_HOFF] = jnp.zeros((C, _HOFF), jnp.float32)
    hflat_ref[:, _HOFF + NP:] = jnp.zeros(
        (C, hflat_ref.shape[1] - _HOFF - NP), jnp.float32)


def _depth_branch_kernel(depth_ref, dflat_ref, hyp_ref, depthn_ref,
                         dflatn_ref, hypn_ref, w1_ref, b1_ref, w2_ref,
                         b2_ref, out_ref, feats0_ref, idx0_ref, hflat0_ref,
                         feats1_ref, idx1_ref, hflat1_ref,
                         *, H, W, C, D, PD, B, R):
    b = pl.program_id(0)
    r = pl.program_id(1)
    j = pl.program_id(2)
    NP = H * _PW
    NCH = NP // _CH
    J = D // PD
    CPB = NCH // R
    RPB = H // R
    NB = B * R

    w1 = w1_ref[...]
    b1c = b1_ref[...].reshape(C, 1)
    b2c = b2_ref[...].reshape(C, 1)
    col = jax.lax.broadcasted_iota(jnp.int32, (1, _CH), 1) % _PW
    vm = col < W

    res = ((dflat_ref, depth_ref, hyp_ref, hflat0_ref, feats0_ref, idx0_ref),
           (dflatn_ref, depthn_ref, hypn_ref, hflat1_ref, feats1_ref,
            idx1_ref))

    def conv1_for(g, jj):
        dfl, _, _, hfl, _, _ = res[g // R]
        c0 = (g % R) * CPB + jj
        _conv1_chunks(dfl[0], hfl, w1, b1c, vm, c0, c0 + 1, C)

    def conv2_for(g, jj):
        _, _, _, hfl, fts, _ = res[g // R]
        c0 = (g % R) * CPB + jj
        _conv2_chunks(hfl, fts, w2_ref, b2c, c0, c0 + 1, C, W)

    def argmin_for(g):
        _, dep, hypr, _, _, idxr = res[g // R]
        y0 = (g % R) * RPB
        _argmin_rows(dep, hypr[0, 0], idxr, y0, y0 + RPB, D, W)

    @pl.when((b == 0) & (r == 0) & (j == 0))
    def _prologue():
        _zero_borders(hflat0_ref, C, NP)
        if B == 2:
            _zero_borders(hflat1_ref, C, NP)
        _conv1_chunks(dflat_ref[0], hflat0_ref, w1, b1c, vm, 0, 2 * CPB, C)
        _conv2_chunks(hflat0_ref, feats0_ref, w2_ref, b2c, 0, CPB, C, W)
        argmin_for(0)

    for bb in range(B):
        for rr in range(R):
            g1 = bb * R + rr + 2
            g2 = bb * R + rr + 1
            for jj in range(J):
                if g1 >= NB and g2 >= NB:
                    continue

                @pl.when((b == bb) & (r == rr) & (j == jj))
                def _piece(g1=g1, g2=g2, jj=jj):
                    if g1 < NB:
                        conv1_for(g1, jj)
                    if g2 < NB:
                        conv2_for(g2, jj)
                        if jj == 2:
                            argmin_for(g2)

    for bb in range(B):
        for rr in range(R):
            @pl.when((b == bb) & (r == rr))
            def _emit(bb=bb, rr=rr):
                _, _, _, _, fts, idxr = res[bb]
                fband = fts[:, rr * RPB:(rr + 1) * RPB, :]
                iband = idxr[rr * RPB:(rr + 1) * RPB, :]
                for p in range(PD):
                    mask = (iband == j * PD + p)[None, :, :]
                    out_ref[0, :, p, :, :] = jnp.where(mask, fband, 0.0)


def kernel(ref_init_depth, depth_hypotheses, W1, b1, W2, b2):
    B, _, H, W = ref_init_depth.shape
    D = depth_hypotheses.shape[1]
    C = W2.shape[0]
    NFLAT = (H + 3) * _PW

    dpad = jnp.pad(ref_init_depth[:, 0], ((0, 0), (1, 2), (1, _PW - W - 1)))
    dflat = dpad.reshape(B, 1, NFLAT)

    w1r = W1.reshape(C, 9)
    w2r = W2.transpose(2, 3, 0, 1).reshape(9, C, C)
    hyp = depth_hypotheses.reshape(B, 1, D)

    PD = 8
    R = 7
    RPB = H // R
    nxt = lambda b, r, d: (jnp.minimum(b + 1, B - 1), 0, 0)
    nxt4 = lambda b, r, d: (jnp.minimum(b + 1, B - 1), 0, 0, 0)
    kfn = functools.partial(_depth_branch_kernel, H=H, W=W, C=C, D=D, PD=PD,
                            B=B, R=R)
    return pl.pallas_call(
        kfn,
        grid=(B, R, D // PD),
        in_specs=[
            pl.BlockSpec((1, 1, H, W), lambda b, r, d: (b, 0, 0, 0)),
            pl.BlockSpec((1, 1, NFLAT), lambda b, r, d: (b, 0, 0)),
            pl.BlockSpec((1, 1, D), lambda b, r, d: (b, 0, 0)),
            pl.BlockSpec((1, 1, H, W), nxt4),
            pl.BlockSpec((1, 1, NFLAT), nxt),
            pl.BlockSpec((1, 1, D), nxt),
            pl.BlockSpec((C, 9), lambda b, r, d: (0, 0)),
            pl.BlockSpec((1, C), lambda b, r, d: (0, 0)),
            pl.BlockSpec((9, C, C), lambda b, r, d: (0, 0, 0)),
            pl.BlockSpec((1, C), lambda b, r, d: (0, 0)),
        ],
        out_specs=pl.BlockSpec((1, C, PD, RPB, W),
                               lambda b, r, d: (b, 0, d, r, 0)),
        out_shape=jax.ShapeDtypeStruct((B, C, D, H, W), jnp.float32),
        scratch_shapes=[
            pltpu.VMEM((C, H, W), jnp.float32),
            pltpu.VMEM((H, W), jnp.int32),
            pltpu.VMEM((C, (H + 4) * _PW), jnp.float32),
            pltpu.VMEM((C, H, W), jnp.float32),
            pltpu.VMEM((H, W), jnp.int32),
            pltpu.VMEM((C, (H + 4) * _PW), jnp.float32),
        ],
        compiler_params=pltpu.CompilerParams(
            dimension_semantics=("arbitrary", "arbitrary", "arbitrary"),
        ),
    )(ref_init_depth, dflat, hyp, ref_init_depth, dflat, hyp, w1r,
      b1.reshape(1, C), w2r, b2.reshape(1, C))


# speedup vs baseline: 15.7121x; 15.7121x over previous
import functools

import jax
import jax.numpy as jnp
from jax.experimental import pallas as pl
from jax.experimental.pallas import tpu as pltpu

_PW = 256
_CH = 1792
_HOFF = 384


def _conv1_piece(dfl, hflat_ref, w1, b1c, vm, c0, c1, C):
    for ci in range(c0, c1):
        n0 = ci * _CH
        acc = jnp.broadcast_to(b1c, (C, _CH))
        for dy in range(3):
            for dx in range(3):
                t = dy * 3 + dx
                off = dy * _PW + dx + n0
                acc = acc + w1[:, t][:, None] * dfl[:, off:off + _CH]
        h = jnp.where(vm, jnp.maximum(acc, 0.0), 0.0)
        hflat_ref[:, _HOFF + n0:_HOFF + n0 + _CH] = h


def _conv2_piece(hflat_ref, feats_ref, w2_ref, b2c, c0, c1, C, W):
    RB = _CH // _PW
    for ci in range(c0, c1):
        n0 = ci * _CH
        facc = None
        for dy in range(3):
            for dx in range(3):
                t = dy * 3 + dx
                off = _HOFF - _PW - 1 + dy * _PW + dx + n0
                dres = jax.lax.dot_general(
                    w2_ref[t], hflat_ref[:, off:off + _CH],
                    (((1,), (0,)), ((), ())),
                    preferred_element_type=jnp.float32)
                facc = dres if facc is None else facc + dres
        feats = jnp.maximum(facc + b2c, 0.0)
        y0 = ci * RB
        feats_ref[:, y0:y0 + RB, :] = feats.reshape(C, RB, _PW)[:, :, :W]


def _argmin_piece(depth_ref, hyp, idx_ref, y0, y1, D, W):
    dch = depth_ref[0, 0, y0:y1, :]
    best = jnp.abs(dch - hyp[0])
    idx = jnp.zeros((y1 - y0, W), jnp.int32)
    for dd in range(1, D):
        diff = jnp.abs(dch - hyp[dd])
        take = diff < best
        best = jnp.where(take, diff, best)
        idx = jnp.where(take, dd, idx)
    idx_ref[y0:y1, :] = idx


def _zero_borders(hflat_ref, C, NP):
    hflat_ref[:, :_HOFF] = jnp.zeros((C, _HOFF), jnp.float32)
    hflat_ref[:, _HOFF + NP:] = jnp.zeros(
        (C, hflat_ref.shape[1] - _HOFF - NP), jnp.float32)


def _depth_branch_kernel(depth_ref, dflat_ref, hyp_ref, depthn_ref,
                         dflatn_ref, hypn_ref, w1_ref, b1_ref, w2_ref,
                         b2_ref, out_ref, feats0_ref, idx0_ref, hflat0_ref,
                         feats1_ref, idx1_ref, hflat1_ref,
                         *, H, W, C, D, PD, B):
    b = pl.program_id(0)
    j = pl.program_id(1)
    NP = H * _PW
    NCH = NP // _CH
    J = D // PD
    CPP = NCH // J
    RPP = H // J

    w1 = w1_ref[...]
    b1c = b1_ref[...].reshape(C, 1)
    b2c = b2_ref[...].reshape(C, 1)
    col = jax.lax.broadcasted_iota(jnp.int32, (1, _CH), 1) % _PW
    vm = col < W

    @pl.when((b == 0) & (j == 0))
    def _prologue():
        dfl = dflat_ref[0]
        hyp = hyp_ref[0, 0]
        _zero_borders(hflat0_ref, C, NP)
        _zero_borders(hflat1_ref, C, NP)
        _conv1_piece(dfl, hflat0_ref, w1, b1c, vm, 0, NCH, C)
        _conv2_piece(hflat0_ref, feats0_ref, w2_ref, b2c, 0, NCH, C, W)
        for k in range(J):
            _argmin_piece(depth_ref, hyp, idx0_ref, k * RPP, (k + 1) * RPP,
                          D, W)

    if B == 2:
        for jj in range(J):
            @pl.when((b == 0) & (j == jj))
            def _piece(jj=jj):
                _conv1_piece(dflatn_ref[0], hflat1_ref, w1, b1c, vm,
                             jj * CPP, (jj + 1) * CPP, C)
                if jj >= 1:
                    _conv2_piece(hflat1_ref, feats1_ref, w2_ref, b2c,
                                 (jj - 1) * CPP, jj * CPP, C, W)
                _argmin_piece(depthn_ref, hypn_ref[0, 0], idx1_ref,
                              jj * RPP, (jj + 1) * RPP, D, W)

        @pl.when((b == 1) & (j == 0))
        def _tail():
            _conv2_piece(hflat1_ref, feats1_ref, w2_ref, b2c,
                         (J - 1) * CPP, NCH, C, W)

    @pl.when(b % 2 == 0)
    def _emit0():
        idx = idx0_ref[...]
        feats = feats0_ref[...]
        for p in range(PD):
            mask = (idx == j * PD + p)[None, :, :]
            out_ref[0, :, p, :, :] = jnp.where(mask, feats, 0.0)

    @pl.when(b % 2 == 1)
    def _emit1():
        idx = idx1_ref[...]
        feats = feats1_ref[...]
        for p in range(PD):
            mask = (idx == j * PD + p)[None, :, :]
            out_ref[0, :, p, :, :] = jnp.where(mask, feats, 0.0)


def kernel(ref_init_depth, depth_hypotheses, W1, b1, W2, b2):
    B, _, H, W = ref_init_depth.shape
    D = depth_hypotheses.shape[1]
    C = W2.shape[0]
    NFLAT = (H + 3) * _PW

    dpad = jnp.pad(ref_init_depth[:, 0], ((0, 0), (1, 2), (1, _PW - W - 1)))
    dflat = dpad.reshape(B, 1, NFLAT)

    w1r = W1.reshape(C, 9)
    w2r = W2.transpose(2, 3, 0, 1).reshape(9, C, C)
    hyp = depth_hypotheses.reshape(B, 1, D)

    PD = 2
    nxt = lambda b, d: (jnp.minimum(b + 1, B - 1), 0, 0)
    nxt4 = lambda b, d: (jnp.minimum(b + 1, B - 1), 0, 0, 0)
    kfn = functools.partial(_depth_branch_kernel, H=H, W=W, C=C, D=D, PD=PD,
                            B=B)
    return pl.pallas_call(
        kfn,
        grid=(B, D // PD),
        in_specs=[
            pl.BlockSpec((1, 1, H, W), lambda b, d: (b, 0, 0, 0)),
            pl.BlockSpec((1, 1, NFLAT), lambda b, d: (b, 0, 0)),
            pl.BlockSpec((1, 1, D), lambda b, d: (b, 0, 0)),
            pl.BlockSpec((1, 1, H, W), nxt4),
            pl.BlockSpec((1, 1, NFLAT), nxt),
            pl.BlockSpec((1, 1, D), nxt),
            pl.BlockSpec((C, 9), lambda b, d: (0, 0)),
            pl.BlockSpec((1, C), lambda b, d: (0, 0)),
            pl.BlockSpec((9, C, C), lambda b, d: (0, 0, 0)),
            pl.BlockSpec((1, C), lambda b, d: (0, 0)),
        ],
        out_specs=pl.BlockSpec((1, C, PD, H, W), lambda b, d: (b, 0, d, 0, 0)),
        out_shape=jax.ShapeDtypeStruct((B, C, D, H, W), jnp.float32),
        scratch_shapes=[
            pltpu.VMEM((C, H, W), jnp.float32),
            pltpu.VMEM((H, W), jnp.int32),
            pltpu.VMEM((C, (H + 4) * _PW), jnp.float32),
            pltpu.VMEM((C, H, W), jnp.float32),
            pltpu.VMEM((H, W), jnp.int32),
            pltpu.VMEM((C, (H + 4) * _PW), jnp.float32),
        ],
        compiler_params=pltpu.CompilerParams(
            dimension_semantics=("arbitrary", "arbitrary"),
        ),
    )(ref_init_depth, dflat, hyp, ref_init_depth, dflat, hyp, w1r,
      b1.reshape(1, C), w2r, b2.reshape(1, C))


# speedup vs baseline: 15.9495x; 1.0151x over previous
import functools

import jax
import jax.numpy as jnp
from jax.experimental import pallas as pl
from jax.experimental.pallas import tpu as pltpu

_PW = 256
_CH = 1792
_HOFF = 384


def _conv1_piece(dfl, hflat_ref, w1, b1c, vm, c0, c1, C):
    for ci in range(c0, c1):
        n0 = ci * _CH
        acc = jnp.broadcast_to(b1c, (C, _CH))
        for dy in range(3):
            for dx in range(3):
                t = dy * 3 + dx
                off = dy * _PW + dx + n0
                acc = acc + w1[:, t][:, None] * dfl[:, off:off + _CH]
        h = jnp.where(vm, jnp.maximum(acc, 0.0), 0.0)
        hflat_ref[:, _HOFF + n0:_HOFF + n0 + _CH] = h


def _conv2_piece(hflat_ref, feats_ref, w2_ref, b2c, c0, c1, C, W):
    RB = _CH // _PW
    for ci in range(c0, c1):
        n0 = ci * _CH
        facc = None
        for dy in range(3):
            for dx in range(3):
                t = dy * 3 + dx
                off = _HOFF - _PW - 1 + dy * _PW + dx + n0
                dres = jax.lax.dot_general(
                    w2_ref[t], hflat_ref[:, off:off + _CH],
                    (((1,), (0,)), ((), ())),
                    preferred_element_type=jnp.float32)
                facc = dres if facc is None else facc + dres
        feats = jnp.maximum(facc + b2c, 0.0)
        y0 = ci * RB
        feats_ref[:, y0:y0 + RB, :] = feats.reshape(C, RB, _PW)[:, :, :W]


def _argmin_piece(depth_ref, hyp, idx_ref, y0, y1, D, W):
    dch = depth_ref[0, 0, y0:y1, :]
    best = jnp.abs(dch - hyp[0])
    idx = jnp.zeros((y1 - y0, W), jnp.int32)
    for dd in range(1, D):
        diff = jnp.abs(dch - hyp[dd])
        take = diff < best
        best = jnp.where(take, diff, best)
        idx = jnp.where(take, dd, idx)
    idx_ref[y0:y1, :] = idx


def _zero_borders(hflat_ref, C, NP):
    hflat_ref[:, :_HOFF] = jnp.zeros((C, _HOFF), jnp.float32)
    hflat_ref[:, _HOFF + NP:] = jnp.zeros(
        (C, hflat_ref.shape[1] - _HOFF - NP), jnp.float32)


def _depth_branch_kernel(depth_ref, dflat_ref, hyp_ref, depthn_ref,
                         dflatn_ref, hypn_ref, w1_ref, b1_ref, w2_ref,
                         b2_ref, out_ref, feats0_ref, idx0_ref, hflat0_ref,
                         feats1_ref, idx1_ref, hflat1_ref,
                         *, H, W, C, D, PD, B):
    b = pl.program_id(0)
    j = pl.program_id(1)
    NP = H * _PW
    NCH = NP // _CH
    J = D // PD
    CPP = NCH // J
    RPP = H // J

    w1 = w1_ref[...]
    b1c = b1_ref[...].reshape(C, 1)
    b2c = b2_ref[...].reshape(C, 1)
    col = jax.lax.broadcasted_iota(jnp.int32, (1, _CH), 1) % _PW
    vm = col < W

    @pl.when((b == 0) & (j == 0))
    def _prologue():
        dfl = dflat_ref[0]
        hyp = hyp_ref[0, 0]
        _zero_borders(hflat0_ref, C, NP)
        _zero_borders(hflat1_ref, C, NP)
        _conv1_piece(dfl, hflat0_ref, w1, b1c, vm, 0, NCH, C)
        _conv2_piece(hflat0_ref, feats0_ref, w2_ref, b2c, 0, NCH, C, W)
        for k in range(J):
            _argmin_piece(depth_ref, hyp, idx0_ref, k * RPP, (k + 1) * RPP,
                          D, W)

    if B == 2:
        for jj in range(J):
            @pl.when((b == 0) & (j == jj))
            def _piece(jj=jj):
                _conv1_piece(dflatn_ref[0], hflat1_ref, w1, b1c, vm,
                             jj * CPP, (jj + 1) * CPP, C)
                if jj >= 1:
                    _conv2_piece(hflat1_ref, feats1_ref, w2_ref, b2c,
                                 (jj - 1) * CPP, jj * CPP, C, W)
                _argmin_piece(depthn_ref, hypn_ref[0, 0], idx1_ref,
                              jj * RPP, (jj + 1) * RPP, D, W)

        @pl.when((b == 1) & (j == 0))
        def _tail():
            _conv2_piece(hflat1_ref, feats1_ref, w2_ref, b2c,
                         (J - 1) * CPP, NCH, C, W)

    EYB = 8

    def _emit(idxr, featsr):
        for y0 in range(0, H, EYB):
            ich = idxr[y0:y0 + EYB, :]
            fch = featsr[:, y0:y0 + EYB, :]
            for p in range(PD):
                mask = (ich == j * PD + p)[None, :, :]
                out_ref[0, :, p, y0:y0 + EYB, :] = jnp.where(mask, fch, 0.0)

    @pl.when(b % 2 == 0)
    def _emit0():
        _emit(idx0_ref, feats0_ref)

    @pl.when(b % 2 == 1)
    def _emit1():
        _emit(idx1_ref, feats1_ref)


def kernel(ref_init_depth, depth_hypotheses, W1, b1, W2, b2):
    B, _, H, W = ref_init_depth.shape
    D = depth_hypotheses.shape[1]
    C = W2.shape[0]
    NFLAT = (H + 3) * _PW

    dpad = jnp.pad(ref_init_depth[:, 0], ((0, 0), (1, 2), (1, _PW - W - 1)))
    dflat = dpad.reshape(B, 1, NFLAT)

    w1r = W1.reshape(C, 9)
    w2r = W2.transpose(2, 3, 0, 1).reshape(9, C, C)
    hyp = depth_hypotheses.reshape(B, 1, D)

    PD = 2
    nxt = lambda b, d: (jnp.minimum(b + 1, B - 1), 0, 0)
    nxt4 = lambda b, d: (jnp.minimum(b + 1, B - 1), 0, 0, 0)
    kfn = functools.partial(_depth_branch_kernel, H=H, W=W, C=C, D=D, PD=PD,
                            B=B)
    return pl.pallas_call(
        kfn,
        grid=(B, D // PD),
        in_specs=[
            pl.BlockSpec((1, 1, H, W), lambda b, d: (b, 0, 0, 0)),
            pl.BlockSpec((1, 1, NFLAT), lambda b, d: (b, 0, 0)),
            pl.BlockSpec((1, 1, D), lambda b, d: (b, 0, 0)),
            pl.BlockSpec((1, 1, H, W), nxt4),
            pl.BlockSpec((1, 1, NFLAT), nxt),
            pl.BlockSpec((1, 1, D), nxt),
            pl.BlockSpec((C, 9), lambda b, d: (0, 0)),
            pl.BlockSpec((1, C), lambda b, d: (0, 0)),
            pl.BlockSpec((9, C, C), lambda b, d: (0, 0, 0)),
            pl.BlockSpec((1, C), lambda b, d: (0, 0)),
        ],
        out_specs=pl.BlockSpec((1, C, PD, H, W), lambda b, d: (b, 0, d, 0, 0)),
        out_shape=jax.ShapeDtypeStruct((B, C, D, H, W), jnp.float32),
        scratch_shapes=[
            pltpu.VMEM((C, H, W), jnp.float32),
            pltpu.VMEM((H, W), jnp.int32),
            pltpu.VMEM((C, (H + 4) * _PW), jnp.float32),
            pltpu.VMEM((C, H, W), jnp.float32),
            pltpu.VMEM((H, W), jnp.int32),
            pltpu.VMEM((C, (H + 4) * _PW), jnp.float32),
        ],
        compiler_params=pltpu.CompilerParams(
            dimension_semantics=("arbitrary", "arbitrary"),
        ),
    )(ref_init_depth, dflat, hyp, ref_init_depth, dflat, hyp, w1r,
      b1.reshape(1, C), w2r, b2.reshape(1, C))


# speedup vs baseline: 17.6665x; 1.1077x over previous
import functools

import jax
import jax.numpy as jnp
from jax.experimental import pallas as pl
from jax.experimental.pallas import tpu as pltpu

_PW = 256
_CH = 1792
_HOFF = 384


def _conv1_piece(dfl, hflat_ref, sconv_ref, w1, b1c, vm, c0, c1, C):
    for ci in range(c0, c1):
        n0 = ci * _CH
        for dy in range(3):
            for dx in range(3):
                t = dy * 3 + dx
                off = dy * _PW + dx + n0
                sconv_ref[t:t + 1, :] = dfl[:, off:off + _CH]
        hh = jax.lax.dot_general(w1, sconv_ref[...], (((1,), (0,)), ((), ())),
                                 preferred_element_type=jnp.float32)
        h = jnp.where(vm, jnp.maximum(hh + b1c, 0.0), 0.0)
        hflat_ref[:, _HOFF + n0:_HOFF + n0 + _CH] = h


def _conv2_piece(hflat_ref, feats_ref, w2_ref, b2c, c0, c1, C, W):
    RB = _CH // _PW
    for ci in range(c0, c1):
        n0 = ci * _CH
        facc = None
        for dy in range(3):
            for dx in range(3):
                t = dy * 3 + dx
                off = _HOFF - _PW - 1 + dy * _PW + dx + n0
                dres = jax.lax.dot_general(
                    w2_ref[t], hflat_ref[:, off:off + _CH],
                    (((1,), (0,)), ((), ())),
                    preferred_element_type=jnp.float32)
                facc = dres if facc is None else facc + dres
        feats = jnp.maximum(facc + b2c, 0.0)
        y0 = ci * RB
        feats_ref[:, y0:y0 + RB, :] = feats.reshape(C, RB, _PW)[:, :, :W]


def _argmin_piece(depth_ref, hyp, idx_ref, y0, y1, D, W):
    dch = depth_ref[0, 0, y0:y1, :]
    best = jnp.abs(dch - hyp[0])
    idx = jnp.zeros((y1 - y0, W), jnp.int32)
    for dd in range(1, D):
        diff = jnp.abs(dch - hyp[dd])
        take = diff < best
        best = jnp.where(take, diff, best)
        idx = jnp.where(take, dd, idx)
    idx_ref[y0:y1, :] = idx


def _zero_borders(hflat_ref, C, NP):
    hflat_ref[:, :_HOFF] = jnp.zeros((C, _HOFF), jnp.float32)
    hflat_ref[:, _HOFF + NP:] = jnp.zeros(
        (C, hflat_ref.shape[1] - _HOFF - NP), jnp.float32)


def _depth_branch_kernel(depth_ref, dflat_ref, hyp_ref, depthn_ref,
                         dflatn_ref, hypn_ref, w1_ref, b1_ref, w2_ref,
                         b2_ref, out_ref, feats0_ref, idx0_ref, hflat0_ref,
                         feats1_ref, idx1_ref, hflat1_ref, sconv_ref,
                         *, H, W, C, D, PD, B):
    b = pl.program_id(0)
    j = pl.program_id(1)
    NP = H * _PW
    NCH = NP // _CH
    J = D // PD
    CPP = NCH // J
    RPP = H // J

    w1 = w1_ref[...]
    b1c = b1_ref[...].reshape(C, 1)
    b2c = b2_ref[...].reshape(C, 1)
    col = jax.lax.broadcasted_iota(jnp.int32, (1, _CH), 1) % _PW
    vm = col < W

    @pl.when((b == 0) & (j == 0))
    def _prologue():
        dfl = dflat_ref[0]
        hyp = hyp_ref[0, 0]
        _zero_borders(hflat0_ref, C, NP)
        _zero_borders(hflat1_ref, C, NP)
        sconv_ref[9:, :] = jnp.zeros((C - 9, _CH), jnp.float32)
        _conv1_piece(dfl, hflat0_ref, sconv_ref, w1, b1c, vm, 0, NCH, C)
        _conv2_piece(hflat0_ref, feats0_ref, w2_ref, b2c, 0, NCH, C, W)
        for k in range(J):
            _argmin_piece(depth_ref, hyp, idx0_ref, k * RPP, (k + 1) * RPP,
                          D, W)

    if B == 2:
        for jj in range(J):
            @pl.when((b == 0) & (j == jj))
            def _piece(jj=jj):
                _conv1_piece(dflatn_ref[0], hflat1_ref, sconv_ref, w1, b1c,
                             vm, jj * CPP, (jj + 1) * CPP, C)
                if jj >= 1:
                    _conv2_piece(hflat1_ref, feats1_ref, w2_ref, b2c,
                                 (jj - 1) * CPP, jj * CPP, C, W)
                _argmin_piece(depthn_ref, hypn_ref[0, 0], idx1_ref,
                              jj * RPP, (jj + 1) * RPP, D, W)

        @pl.when((b == 1) & (j == 0))
        def _tail():
            _conv2_piece(hflat1_ref, feats1_ref, w2_ref, b2c,
                         (J - 1) * CPP, NCH, C, W)

    EYB = 8

    def _emit(idxr, featsr):
        for y0 in range(0, H, EYB):
            ich = idxr[y0:y0 + EYB, :]
            fch = featsr[:, y0:y0 + EYB, :]
            for p in range(PD):
                mask = (ich == j * PD + p)[None, :, :]
                out_ref[0, :, p, y0:y0 + EYB, :] = jnp.where(mask, fch, 0.0)

    @pl.when(b % 2 == 0)
    def _emit0():
        _emit(idx0_ref, feats0_ref)

    @pl.when(b % 2 == 1)
    def _emit1():
        _emit(idx1_ref, feats1_ref)


def kernel(ref_init_depth, depth_hypotheses, W1, b1, W2, b2):
    B, _, H, W = ref_init_depth.shape
    D = depth_hypotheses.shape[1]
    C = W2.shape[0]
    NFLAT = (H + 3) * _PW

    dpad = jnp.pad(ref_init_depth[:, 0], ((0, 0), (1, 2), (1, _PW - W - 1)))
    dflat = dpad.reshape(B, 1, NFLAT)

    w1r = jnp.pad(W1.reshape(C, 9), ((0, 0), (0, C - 9)))
    w2r = W2.transpose(2, 3, 0, 1).reshape(9, C, C)
    hyp = depth_hypotheses.reshape(B, 1, D)

    PD = 2
    nxt = lambda b, d: (jnp.minimum(b + 1, B - 1), 0, 0)
    nxt4 = lambda b, d: (jnp.minimum(b + 1, B - 1), 0, 0, 0)
    kfn = functools.partial(_depth_branch_kernel, H=H, W=W, C=C, D=D, PD=PD,
                            B=B)
    return pl.pallas_call(
        kfn,
        grid=(B, D // PD),
        in_specs=[
            pl.BlockSpec((1, 1, H, W), lambda b, d: (b, 0, 0, 0)),
            pl.BlockSpec((1, 1, NFLAT), lambda b, d: (b, 0, 0)),
            pl.BlockSpec((1, 1, D), lambda b, d: (b, 0, 0)),
            pl.BlockSpec((1, 1, H, W), nxt4),
            pl.BlockSpec((1, 1, NFLAT), nxt),
            pl.BlockSpec((1, 1, D), nxt),
            pl.BlockSpec((C, C), lambda b, d: (0, 0)),
            pl.BlockSpec((1, C), lambda b, d: (0, 0)),
            pl.BlockSpec((9, C, C), lambda b, d: (0, 0, 0)),
            pl.BlockSpec((1, C), lambda b, d: (0, 0)),
        ],
        out_specs=pl.BlockSpec((1, C, PD, H, W), lambda b, d: (b, 0, d, 0, 0)),
        out_shape=jax.ShapeDtypeStruct((B, C, D, H, W), jnp.float32),
        scratch_shapes=[
            pltpu.VMEM((C, H, W), jnp.float32),
            pltpu.VMEM((H, W), jnp.int32),
            pltpu.VMEM((C, (H + 4) * _PW), jnp.float32),
            pltpu.VMEM((C, H, W), jnp.float32),
            pltpu.VMEM((H, W), jnp.int32),
            pltpu.VMEM((C, (H + 4) * _PW), jnp.float32),
            pltpu.VMEM((C, _CH), jnp.float32),
        ],
        compiler_params=pltpu.CompilerParams(
            dimension_semantics=("arbitrary", "arbitrary"),
        ),
    )(ref_init_depth, dflat, hyp, ref_init_depth, dflat, hyp, w1r,
      b1.reshape(1, C), w2r, b2.reshape(1, C))
